# baseline (speedup 1.0000x reference)
import jax
import jax.numpy as jnp
from jax.experimental import pallas as pl
from jax.experimental.pallas import tpu as pltpu

B, N, C, K_NN = 32, 256, 128, 20
LANG = 256
HID = 128
BN_ALL = B * N
EPS = 1e-5
F32 = jnp.float32
BF16 = jnp.bfloat16


def _mm_t(a, w):
    return jax.lax.dot_general(a.astype(BF16), w.astype(BF16),
                               (((1,), (1,)), ((), ())),
                               preferred_element_type=F32)


def _mm(a, b):
    return jax.lax.dot_general(a.astype(BF16), b.astype(BF16),
                               (((1,), (0,)), ((), ())),
                               preferred_element_type=F32)


def _colsum(v):
    return jnp.sum(v, axis=0, keepdims=True)


def _fused_body(feats_ref, obj_ref, lang_ref,
                graph_W_ref, graph_b_ref, graph_g_ref, graph_be_ref,
                fuse_W_ref, fuse_b_ref,
                w1_ref, b1_ref, g1_ref, be1_ref,
                w2_ref, b2_ref, g2_ref, be2_ref,
                w3_ref, b3_ref,
                out_ref, scr_ref, oh_ref):
    jrow_f = jax.lax.broadcasted_iota(jnp.int32, (1, N), 1).astype(F32)
    eye = (jax.lax.broadcasted_iota(jnp.int32, (N, N), 1)
           == jax.lax.broadcasted_iota(jnp.int32, (N, N), 0))
    Wn = graph_W_ref[:, :C]
    Wc = graph_W_ref[:, C:]
    gb = graph_b_ref[...]
    BIG = jnp.float32(2 * N)

    def p1_body(b, carry):
        S1, Q1 = carry
        x = feats_ref[pl.ds(b, 1)].reshape(N, C)
        x_hi = x.astype(BF16).astype(F32)
        x_lo = (x - x_hi).astype(BF16)
        x2b = jnp.concatenate([x_hi.astype(BF16), x_lo], axis=1)
        xxT = _mm_t(x, x)
        sq_col = jnp.sum(x * x, axis=1, keepdims=True)
        sq_row = jnp.sum(jnp.where(eye, sq_col, 0.0), axis=0, keepdims=True)
        score0 = sq_row - 2.0 * xxT
        P_c = _mm_t(x, Wc) + gb
        Wn_b = Wn.astype(BF16)

        def select(score):
            m = jnp.min(score, axis=1, keepdims=True)
            cand = jnp.where(score <= m, jrow_f, BIG)
            jmin = jnp.min(cand, axis=1, keepdims=True)
            onehot = jrow_f == jmin
            return onehot.astype(BF16), jnp.where(onehot, jnp.inf, score)

        def accum(onehot_b, M, Sa, Qa):
            nbr2 = jax.lax.dot_general(onehot_b, x2b, (((1,), (0,)), ((), ())),
                                       preferred_element_type=F32)
            nbr = nbr2[:, :C] + nbr2[:, C:]
            edge = (nbr - x).astype(BF16)
            h_k = jax.lax.dot_general(edge, Wn_b, (((1,), (1,)), ((), ())),
                                      preferred_element_type=F32) + P_c
            return jnp.maximum(M, h_k), Sa + h_k, Qa + h_k * h_k

        def sel_body(k, score):
            onehot_b, score_n = select(score)
            oh_ref[pl.ds(k, 1)] = onehot_b.reshape(1, N, N)
            return score_n

        jax.lax.fori_loop(0, K_NN, sel_body, score0)

        def acc_body(k, kc):
            M, Sa, Qa = kc
            return accum(oh_ref[pl.ds(k, 1)].reshape(N, N), M, Sa, Qa)

        M0 = jnp.full((N, HID), -jnp.inf, F32)
        Z = jnp.zeros((N, HID), F32)
        M, Sa, Qa = jax.lax.fori_loop(0, K_NN, acc_body, (M0, Z, Z))
        scr_ref[pl.ds(b, 1)] = M.reshape(1, N, HID)
        return S1 + _colsum(Sa), Q1 + _colsum(Qa)

    zero_row = jnp.zeros((1, HID), F32)
    S1, Q1 = jax.lax.fori_loop(0, B, p1_body, (zero_row, zero_row))

    cnt1 = float(B * N * K_NN)
    mean1 = S1 / cnt1
    var1 = Q1 / cnt1 - mean1 * mean1
    sc1 = graph_g_ref[...] / jnp.sqrt(var1 + EPS)
    sh1 = graph_be_ref[...] - mean1 * sc1

    Wf_g = fuse_W_ref[:, :HID]
    Wf_l = fuse_W_ref[:, HID:]

    Mall = scr_ref[...].reshape(BN_ALL, HID)
    t = Mall * sc1 + sh1
    go = jnp.maximum(t, 0.2 * t)
    lf = _mm_t(lang_ref[...], Wf_l)
    lf_exp = jnp.broadcast_to(lf.reshape(B, 1, HID),
                              (B, N, HID)).reshape(BN_ALL, HID)
    ob = obj_ref[...].reshape(BN_ALL, 2)
    maskf = (ob[:, 1:2] > ob[:, 0:1]).astype(F32)
    f = jnp.maximum(_mm_t(go, Wf_g) + lf_exp + fuse_b_ref[...], 0.0) * maskf
    m1 = jnp.maximum(_mm_t(f, w1_ref[...]) + b1_ref[...], 0.0)

    cnt2 = float(BN_ALL)
    mean2 = _colsum(m1) / cnt2
    var2 = _colsum(m1 * m1) / cnt2 - mean2 * mean2
    sc2 = g1_ref[...] / jnp.sqrt(var2 + EPS)
    sh2 = be1_ref[...] - mean2 * sc2

    m2 = jnp.maximum(_mm_t(m1 * sc2 + sh2, w2_ref[...]) + b2_ref[...], 0.0)
    mean3 = _colsum(m2) / cnt2
    var3 = _colsum(m2 * m2) / cnt2 - mean3 * mean3
    sc3 = g2_ref[...] / jnp.sqrt(var3 + EPS)
    sh3 = be2_ref[...] - mean3 * sc3

    m2n = m2 * sc3 + sh3
    conf = _mm_t(w3_ref[...], m2n) + b3_ref[0, 0]
    out_ref[...] = conf.reshape(B, N)


def kernel(aggregated_vote_features, objectness_scores, lang_emb,
           graph_W, graph_b, graph_gamma, graph_beta,
           fuse_W, fuse_b,
           match_W1, match_b1, match_g1, match_be1,
           match_W2, match_b2, match_g2, match_be2,
           match_W3, match_b3):
    row = lambda v: v.reshape(1, -1)
    return pl.pallas_call(
        _fused_body,
        out_shape=jax.ShapeDtypeStruct((B, N), F32),
        scratch_shapes=[pltpu.VMEM((B, N, HID), F32),
                        pltpu.VMEM((K_NN, N, N), BF16)],
    )(aggregated_vote_features, objectness_scores, lang_emb,
      graph_W, row(graph_b), row(graph_gamma), row(graph_beta),
      fuse_W, row(fuse_b),
      match_W1, row(match_b1), row(match_g1), row(match_be1),
      match_W2, row(match_b2), row(match_g2), row(match_be2),
      match_W3, row(match_b3))


# speedup vs baseline: 1.8532x; 1.8532x over previous
import jax
import jax.numpy as jnp
from jax.experimental import pallas as pl
from jax.experimental.pallas import tpu as pltpu

B, N, C, K_NN = 32, 256, 128, 20
LANG = 256
HID = 128
BN_ALL = B * N
EPS = 1e-5
F32 = jnp.float32
BF16 = jnp.bfloat16


def _mm_t(a, w):
    return jax.lax.dot_general(a.astype(BF16), w.astype(BF16),
                               (((1,), (1,)), ((), ())),
                               preferred_element_type=F32)


def _mm(a, b):
    return jax.lax.dot_general(a.astype(BF16), b.astype(BF16),
                               (((1,), (0,)), ((), ())),
                               preferred_element_type=F32)


def _colsum(v):
    return jnp.sum(v, axis=0, keepdims=True)


def _fused_body(feats_ref, obj_ref, lang_ref,
                graph_W_ref, graph_b_ref, graph_g_ref, graph_be_ref,
                fuse_W_ref, fuse_b_ref,
                w1_ref, b1_ref, g1_ref, be1_ref,
                w2_ref, b2_ref, g2_ref, be2_ref,
                w3_ref, b3_ref,
                out_ref, scr_ref):
    jrow_f = jax.lax.broadcasted_iota(jnp.int32, (1, N), 1).astype(F32)
    eye = (jax.lax.broadcasted_iota(jnp.int32, (N, N), 1)
           == jax.lax.broadcasted_iota(jnp.int32, (N, N), 0))
    Wn = graph_W_ref[:, :C]
    Wc = graph_W_ref[:, C:]
    gb = graph_b_ref[...]
    BIG = jnp.float32(2 * N)

    def p1_body(b, carry):
        S1, Q1 = carry
        x = feats_ref[pl.ds(b, 1)].reshape(N, C)
        x_hi = x.astype(BF16).astype(F32)
        x_lo = (x - x_hi).astype(BF16)
        x2b = jnp.concatenate([x_hi.astype(BF16), x_lo], axis=1)
        xxT = _mm_t(x, x)
        sq_col = jnp.sum(x * x, axis=1, keepdims=True)
        sq_row = jnp.sum(jnp.where(eye, sq_col, 0.0), axis=0, keepdims=True)
        score0 = sq_row - 2.0 * xxT
        P_c = _mm_t(x, Wc) + gb
        Wn_b = Wn.astype(BF16)

        def select(score):
            m = jnp.min(score, axis=1, keepdims=True)
            cand = jnp.where(score <= m, jrow_f, BIG)
            jmin = jnp.min(cand, axis=1, keepdims=True)
            onehot = jrow_f == jmin
            return onehot.astype(BF16), jnp.where(onehot, jnp.inf, score)

        def accum2(ohA, ohB, M, Sa, Qa):
            oh2 = jnp.concatenate([ohA, ohB], axis=0)
            nbr2 = jax.lax.dot_general(oh2, x2b, (((1,), (0,)), ((), ())),
                                       preferred_element_type=F32)
            nbr = nbr2[:, :C] + nbr2[:, C:]
            edge_a = (nbr[:N] - x).astype(BF16)
            edge_b = (nbr[N:] - x).astype(BF16)
            h_a = jax.lax.dot_general(edge_a, Wn_b, (((1,), (1,)), ((), ())),
                                      preferred_element_type=F32) + P_c
            h_b = jax.lax.dot_general(edge_b, Wn_b, (((1,), (1,)), ((), ())),
                                      preferred_element_type=F32) + P_c
            M = jnp.maximum(M, jnp.maximum(h_a, h_b))
            return M, Sa + (h_a + h_b), Qa + (h_a * h_a + h_b * h_b)

        def k_body(_, kc):
            score, ohA, ohB, M, Sa, Qa = kc
            M, Sa, Qa = accum2(ohA, ohB, M, Sa, Qa)
            ohA_n, score = select(score)
            ohB_n, score = select(score)
            return score, ohA_n, ohB_n, M, Sa, Qa

        M0 = jnp.full((N, HID), -jnp.inf, F32)
        Z = jnp.zeros((N, HID), F32)
        ohA0, score1 = select(score0)
        ohB0, score2 = select(score1)
        _, ohA_l, ohB_l, M, Sa, Qa = jax.lax.fori_loop(
            0, K_NN // 2 - 1, k_body, (score2, ohA0, ohB0, M0, Z, Z))
        M, Sa, Qa = accum2(ohA_l, ohB_l, M, Sa, Qa)
        scr_ref[pl.ds(b, 1)] = M.reshape(1, N, HID)
        return S1 + _colsum(Sa), Q1 + _colsum(Qa)

    zero_row = jnp.zeros((1, HID), F32)
    S1, Q1 = jax.lax.fori_loop(0, B, p1_body, (zero_row, zero_row))

    cnt1 = float(B * N * K_NN)
    mean1 = S1 / cnt1
    var1 = Q1 / cnt1 - mean1 * mean1
    sc1 = graph_g_ref[...] / jnp.sqrt(var1 + EPS)
    sh1 = graph_be_ref[...] - mean1 * sc1

    Wf_g = fuse_W_ref[:, :HID]
    Wf_l = fuse_W_ref[:, HID:]

    Mall = scr_ref[...].reshape(BN_ALL, HID)
    t = Mall * sc1 + sh1
    go = jnp.maximum(t, 0.2 * t)
    lf = _mm_t(lang_ref[...], Wf_l)
    lf_exp = jnp.broadcast_to(lf.reshape(B, 1, HID),
                              (B, N, HID)).reshape(BN_ALL, HID)
    ob = obj_ref[...].reshape(BN_ALL, 2)
    maskf = (ob[:, 1:2] > ob[:, 0:1]).astype(F32)
    f = jnp.maximum(_mm_t(go, Wf_g) + lf_exp + fuse_b_ref[...], 0.0) * maskf
    m1 = jnp.maximum(_mm_t(f, w1_ref[...]) + b1_ref[...], 0.0)

    cnt2 = float(BN_ALL)
    mean2 = _colsum(m1) / cnt2
    var2 = _colsum(m1 * m1) / cnt2 - mean2 * mean2
    sc2 = g1_ref[...] / jnp.sqrt(var2 + EPS)
    sh2 = be1_ref[...] - mean2 * sc2

    m2 = jnp.maximum(_mm_t(m1 * sc2 + sh2, w2_ref[...]) + b2_ref[...], 0.0)
    mean3 = _colsum(m2) / cnt2
    var3 = _colsum(m2 * m2) / cnt2 - mean3 * mean3
    sc3 = g2_ref[...] / jnp.sqrt(var3 + EPS)
    sh3 = be2_ref[...] - mean3 * sc3

    m2n = m2 * sc3 + sh3
    conf = _mm_t(w3_ref[...], m2n) + b3_ref[0, 0]
    out_ref[...] = conf.reshape(B, N)


def kernel(aggregated_vote_features, objectness_scores, lang_emb,
           graph_W, graph_b, graph_gamma, graph_beta,
           fuse_W, fuse_b,
           match_W1, match_b1, match_g1, match_be1,
           match_W2, match_b2, match_g2, match_be2,
           match_W3, match_b3):
    row = lambda v: v.reshape(1, -1)
    return pl.pallas_call(
        _fused_body,
        out_shape=jax.ShapeDtypeStruct((B, N), F32),
        scratch_shapes=[pltpu.VMEM((B, N, HID), F32)],
    )(aggregated_vote_features, objectness_scores, lang_emb,
      graph_W, row(graph_b), row(graph_gamma), row(graph_beta),
      fuse_W, row(fuse_b),
      match_W1, row(match_b1), row(match_g1), row(match_be1),
      match_W2, row(match_b2), row(match_g2), row(match_be2),
      match_W3, row(match_b3))


# speedup vs baseline: 2.0872x; 1.1263x over previous
import jax
import jax.numpy as jnp
from jax.experimental import pallas as pl
from jax.experimental.pallas import tpu as pltpu

B, N, C, K_NN = 32, 256, 128, 20
LANG = 256
HID = 128
BN_ALL = B * N
EPS = 1e-5
F32 = jnp.float32
BF16 = jnp.bfloat16


def _mm_t(a, w):
    return jax.lax.dot_general(a.astype(BF16), w.astype(BF16),
                               (((1,), (1,)), ((), ())),
                               preferred_element_type=F32)


def _mm(a, b):
    return jax.lax.dot_general(a.astype(BF16), b.astype(BF16),
                               (((1,), (0,)), ((), ())),
                               preferred_element_type=F32)


def _colsum(v):
    return jnp.sum(v, axis=0, keepdims=True)


def _fused_body(feats_ref, obj_ref, lang_ref,
                graph_W_ref, graph_b_ref, graph_g_ref, graph_be_ref,
                fuse_W_ref, fuse_b_ref,
                w1_ref, b1_ref, g1_ref, be1_ref,
                w2_ref, b2_ref, g2_ref, be2_ref,
                w3_ref, b3_ref,
                out_ref, scr_ref):
    jrow_f = jax.lax.broadcasted_iota(jnp.int32, (1, N), 1).astype(F32)
    eye = (jax.lax.broadcasted_iota(jnp.int32, (N, N), 1)
           == jax.lax.broadcasted_iota(jnp.int32, (N, N), 0))
    Wn = graph_W_ref[:, :C]
    Wc = graph_W_ref[:, C:]
    gb = graph_b_ref[...]
    BIG = jnp.float32(2 * N)

    def p1_body(b, carry):
        S1, Q1 = carry
        x = feats_ref[pl.ds(b, 1)].reshape(N, C)
        x_hi = x.astype(BF16).astype(F32)
        x_lo = (x - x_hi).astype(BF16)
        x2b = jnp.concatenate([x_hi.astype(BF16), x_lo], axis=1)
        xxT = _mm_t(x, x)
        sq_col = jnp.sum(x * x, axis=1, keepdims=True)
        sq_row = jnp.sum(jnp.where(eye, sq_col, 0.0), axis=0, keepdims=True)
        score0 = sq_row - 2.0 * xxT
        P_c = _mm_t(x, Wc) + gb
        Wn_b = Wn.astype(BF16)

        def select(score):
            m = jnp.min(score, axis=1, keepdims=True)
            cand = jnp.where(score <= m, jrow_f, BIG)
            jmin = jnp.min(cand, axis=1, keepdims=True)
            onehot = jrow_f == jmin
            return onehot.astype(BF16), jnp.where(onehot, jnp.inf, score)

        def accum4(ohs, M, Sa, Qa):
            oh4 = jnp.concatenate(ohs, axis=0)
            nbr2 = jax.lax.dot_general(oh4, x2b, (((1,), (0,)), ((), ())),
                                       preferred_element_type=F32)
            nbr = nbr2[:, :C] + nbr2[:, C:]
            hs = []
            for i in range(4):
                edge = (nbr[i * N:(i + 1) * N] - x).astype(BF16)
                hs.append(jax.lax.dot_general(
                    edge, Wn_b, (((1,), (1,)), ((), ())),
                    preferred_element_type=F32) + P_c)
            M = jnp.maximum(M, jnp.maximum(jnp.maximum(hs[0], hs[1]),
                                           jnp.maximum(hs[2], hs[3])))
            Sa = Sa + ((hs[0] + hs[1]) + (hs[2] + hs[3]))
            Qa = Qa + ((hs[0] * hs[0] + hs[1] * hs[1])
                       + (hs[2] * hs[2] + hs[3] * hs[3]))
            return M, Sa, Qa

        def select4(score):
            ohs = []
            for _ in range(4):
                oh, score = select(score)
                ohs.append(oh)
            return ohs, score

        def k_body(_, kc):
            score, oh0, oh1, oh2, oh3, M, Sa, Qa = kc
            M, Sa, Qa = accum4((oh0, oh1, oh2, oh3), M, Sa, Qa)
            ohs_n, score = select4(score)
            return (score, *ohs_n, M, Sa, Qa)

        M0 = jnp.full((N, HID), -jnp.inf, F32)
        Z = jnp.zeros((N, HID), F32)
        ohs0, score1 = select4(score0)
        out_c = jax.lax.fori_loop(
            0, K_NN // 4 - 1, k_body, (score1, *ohs0, M0, Z, Z))
        M, Sa, Qa = accum4(out_c[1:5], *out_c[5:])
        scr_ref[pl.ds(b, 1)] = M.reshape(1, N, HID)
        return S1 + _colsum(Sa), Q1 + _colsum(Qa)

    zero_row = jnp.zeros((1, HID), F32)
    S1, Q1 = jax.lax.fori_loop(0, B, p1_body, (zero_row, zero_row))

    cnt1 = float(B * N * K_NN)
    mean1 = S1 / cnt1
    var1 = Q1 / cnt1 - mean1 * mean1
    sc1 = graph_g_ref[...] / jnp.sqrt(var1 + EPS)
    sh1 = graph_be_ref[...] - mean1 * sc1

    Wf_g = fuse_W_ref[:, :HID]
    Wf_l = fuse_W_ref[:, HID:]

    Mall = scr_ref[...].reshape(BN_ALL, HID)
    t = Mall * sc1 + sh1
    go = jnp.maximum(t, 0.2 * t)
    lf = _mm_t(lang_ref[...], Wf_l)
    lf_exp = jnp.broadcast_to(lf.reshape(B, 1, HID),
                              (B, N, HID)).reshape(BN_ALL, HID)
    ob = obj_ref[...].reshape(BN_ALL, 2)
    maskf = (ob[:, 1:2] > ob[:, 0:1]).astype(F32)
    f = jnp.maximum(_mm_t(go, Wf_g) + lf_exp + fuse_b_ref[...], 0.0) * maskf
    m1 = jnp.maximum(_mm_t(f, w1_ref[...]) + b1_ref[...], 0.0)

    cnt2 = float(BN_ALL)
    mean2 = _colsum(m1) / cnt2
    var2 = _colsum(m1 * m1) / cnt2 - mean2 * mean2
    sc2 = g1_ref[...] / jnp.sqrt(var2 + EPS)
    sh2 = be1_ref[...] - mean2 * sc2

    m2 = jnp.maximum(_mm_t(m1 * sc2 + sh2, w2_ref[...]) + b2_ref[...], 0.0)
    mean3 = _colsum(m2) / cnt2
    var3 = _colsum(m2 * m2) / cnt2 - mean3 * mean3
    sc3 = g2_ref[...] / jnp.sqrt(var3 + EPS)
    sh3 = be2_ref[...] - mean3 * sc3

    m2n = m2 * sc3 + sh3
    conf = _mm_t(w3_ref[...], m2n) + b3_ref[0, 0]
    out_ref[...] = conf.reshape(B, N)


def kernel(aggregated_vote_features, objectness_scores, lang_emb,
           graph_W, graph_b, graph_gamma, graph_beta,
           fuse_W, fuse_b,
           match_W1, match_b1, match_g1, match_be1,
           match_W2, match_b2, match_g2, match_be2,
           match_W3, match_b3):
    row = lambda v: v.reshape(1, -1)
    return pl.pallas_call(
        _fused_body,
        out_shape=jax.ShapeDtypeStruct((B, N), F32),
        scratch_shapes=[pltpu.VMEM((B, N, HID), F32)],
    )(aggregated_vote_features, objectness_scores, lang_emb,
      graph_W, row(graph_b), row(graph_gamma), row(graph_beta),
      fuse_W, row(fuse_b),
      match_W1, row(match_b1), row(match_g1), row(match_be1),
      match_W2, row(match_b2), row(match_g2), row(match_be2),
      match_W3, row(match_b3))


# speedup vs baseline: 2.7358x; 1.3107x over previous
import jax
import jax.numpy as jnp
from jax.experimental import pallas as pl
from jax.experimental.pallas import tpu as pltpu

B, N, C, K_NN = 32, 256, 128, 20
LANG = 256
HID = 128
BN_ALL = B * N
EPS = 1e-5
F32 = jnp.float32
BF16 = jnp.bfloat16


def _mm_t(a, w):
    return jax.lax.dot_general(a.astype(BF16), w.astype(BF16),
                               (((1,), (1,)), ((), ())),
                               preferred_element_type=F32)


def _mm(a, b):
    return jax.lax.dot_general(a.astype(BF16), b.astype(BF16),
                               (((1,), (0,)), ((), ())),
                               preferred_element_type=F32)


def _colsum(v):
    return jnp.sum(v, axis=0, keepdims=True)


def _fused_body(feats_ref, obj_ref, lang_ref,
                graph_W_ref, graph_b_ref, graph_g_ref, graph_be_ref,
                fuse_W_ref, fuse_b_ref,
                w1_ref, b1_ref, g1_ref, be1_ref,
                w2_ref, b2_ref, g2_ref, be2_ref,
                w3_ref, b3_ref,
                out_ref, scr_ref):
    jrow_f = jax.lax.broadcasted_iota(jnp.int32, (1, N), 1).astype(F32)
    eye = (jax.lax.broadcasted_iota(jnp.int32, (N, N), 1)
           == jax.lax.broadcasted_iota(jnp.int32, (N, N), 0))
    Wn = graph_W_ref[:, :C]
    Wc = graph_W_ref[:, C:]
    gb = graph_b_ref[...]
    BIG = jnp.float32(2 * N)

    def p1_body(b, carry):
        S1, Q1 = carry
        x = feats_ref[pl.ds(b, 1)].reshape(N, C)
        x_hi = x.astype(BF16).astype(F32)
        x_lo = (x - x_hi).astype(BF16)
        x2b = jnp.concatenate([x_hi.astype(BF16), x_lo], axis=1)
        xxT = _mm_t(x, x)
        sq_col = jnp.sum(x * x, axis=1, keepdims=True)
        sq_row = jnp.sum(jnp.where(eye, sq_col, 0.0), axis=0, keepdims=True)
        score0 = sq_row - 2.0 * xxT
        P_c = _mm_t(x, Wc) + gb
        Wn_b = Wn.astype(BF16)

        def select(score):
            m = jnp.min(score, axis=1, keepdims=True)
            cand = jnp.where(score <= m, jrow_f, BIG)
            jmin = jnp.min(cand, axis=1, keepdims=True)
            onehot = jrow_f == jmin
            return onehot.astype(BF16), jnp.where(onehot, jnp.inf, score)

        def accum4(ohs, M, Sa, Qa):
            oh4 = jnp.concatenate(ohs, axis=0)
            nbr2 = jax.lax.dot_general(oh4, x2b, (((1,), (0,)), ((), ())),
                                       preferred_element_type=F32)
            nbr = nbr2[:, :C] + nbr2[:, C:]
            hs = []
            for i in range(4):
                edge = (nbr[i * N:(i + 1) * N] - x).astype(BF16)
                hs.append(jax.lax.dot_general(
                    edge, Wn_b, (((1,), (1,)), ((), ())),
                    preferred_element_type=F32) + P_c)
            M = jnp.maximum(M, jnp.maximum(jnp.maximum(hs[0], hs[1]),
                                           jnp.maximum(hs[2], hs[3])))
            Sa = Sa + ((hs[0] + hs[1]) + (hs[2] + hs[3]))
            Qa = Qa + ((hs[0] * hs[0] + hs[1] * hs[1])
                       + (hs[2] * hs[2] + hs[3] * hs[3]))
            return M, Sa, Qa

        def select4(score):
            ohs = []
            for _ in range(4):
                oh, score = select(score)
                ohs.append(oh)
            return ohs, score

        M = jnp.full((N, HID), -jnp.inf, F32)
        Sa = jnp.zeros((N, HID), F32)
        Qa = jnp.zeros((N, HID), F32)
        score = score0
        for _ in range(K_NN // 4):
            ohs, score = select4(score)
            M, Sa, Qa = accum4(ohs, M, Sa, Qa)
        scr_ref[pl.ds(b, 1)] = M.reshape(1, N, HID)
        return S1 + _colsum(Sa), Q1 + _colsum(Qa)

    zero_row = jnp.zeros((1, HID), F32)
    S1, Q1 = jax.lax.fori_loop(0, B, p1_body, (zero_row, zero_row))

    cnt1 = float(B * N * K_NN)
    mean1 = S1 / cnt1
    var1 = Q1 / cnt1 - mean1 * mean1
    sc1 = graph_g_ref[...] / jnp.sqrt(var1 + EPS)
    sh1 = graph_be_ref[...] - mean1 * sc1

    Wf_g = fuse_W_ref[:, :HID]
    Wf_l = fuse_W_ref[:, HID:]

    Mall = scr_ref[...].reshape(BN_ALL, HID)
    t = Mall * sc1 + sh1
    go = jnp.maximum(t, 0.2 * t)
    lf = _mm_t(lang_ref[...], Wf_l)
    lf_exp = jnp.broadcast_to(lf.reshape(B, 1, HID),
                              (B, N, HID)).reshape(BN_ALL, HID)
    ob = obj_ref[...].reshape(BN_ALL, 2)
    maskf = (ob[:, 1:2] > ob[:, 0:1]).astype(F32)
    f = jnp.maximum(_mm_t(go, Wf_g) + lf_exp + fuse_b_ref[...], 0.0) * maskf
    m1 = jnp.maximum(_mm_t(f, w1_ref[...]) + b1_ref[...], 0.0)

    cnt2 = float(BN_ALL)
    mean2 = _colsum(m1) / cnt2
    var2 = _colsum(m1 * m1) / cnt2 - mean2 * mean2
    sc2 = g1_ref[...] / jnp.sqrt(var2 + EPS)
    sh2 = be1_ref[...] - mean2 * sc2

    m2 = jnp.maximum(_mm_t(m1 * sc2 + sh2, w2_ref[...]) + b2_ref[...], 0.0)
    mean3 = _colsum(m2) / cnt2
    var3 = _colsum(m2 * m2) / cnt2 - mean3 * mean3
    sc3 = g2_ref[...] / jnp.sqrt(var3 + EPS)
    sh3 = be2_ref[...] - mean3 * sc3

    m2n = m2 * sc3 + sh3
    conf = _mm_t(w3_ref[...], m2n) + b3_ref[0, 0]
    out_ref[...] = conf.reshape(B, N)


def kernel(aggregated_vote_features, objectness_scores, lang_emb,
           graph_W, graph_b, graph_gamma, graph_beta,
           fuse_W, fuse_b,
           match_W1, match_b1, match_g1, match_be1,
           match_W2, match_b2, match_g2, match_be2,
           match_W3, match_b3):
    row = lambda v: v.reshape(1, -1)
    return pl.pallas_call(
        _fused_body,
        out_shape=jax.ShapeDtypeStruct((B, N), F32),
        scratch_shapes=[pltpu.VMEM((B, N, HID), F32)],
    )(aggregated_vote_features, objectness_scores, lang_emb,
      graph_W, row(graph_b), row(graph_gamma), row(graph_beta),
      fuse_W, row(fuse_b),
      match_W1, row(match_b1), row(match_g1), row(match_be1),
      match_W2, row(match_b2), row(match_g2), row(match_be2),
      match_W3, row(match_b3))


# speedup vs baseline: 2.7788x; 1.0157x over previous
import jax
import jax.numpy as jnp
from jax.experimental import pallas as pl
from jax.experimental.pallas import tpu as pltpu

B, N, C, K_NN = 32, 256, 128, 20
LANG = 256
HID = 128
BN_ALL = B * N
EPS = 1e-5
F32 = jnp.float32
BF16 = jnp.bfloat16


def _mm_t(a, w):
    return jax.lax.dot_general(a.astype(BF16), w.astype(BF16),
                               (((1,), (1,)), ((), ())),
                               preferred_element_type=F32)


def _mm(a, b):
    return jax.lax.dot_general(a.astype(BF16), b.astype(BF16),
                               (((1,), (0,)), ((), ())),
                               preferred_element_type=F32)


def _colsum(v):
    return jnp.sum(v, axis=0, keepdims=True)


def _fused_body(feats_ref, obj_ref, lang_ref,
                graph_W_ref, graph_b_ref, graph_g_ref, graph_be_ref,
                fuse_W_ref, fuse_b_ref,
                w1_ref, b1_ref, g1_ref, be1_ref,
                w2_ref, b2_ref, g2_ref, be2_ref,
                w3_ref, b3_ref,
                out_ref, scr_ref):
    jrow_f = jax.lax.broadcasted_iota(jnp.int32, (1, N), 1).astype(F32)
    eye = (jax.lax.broadcasted_iota(jnp.int32, (N, N), 1)
           == jax.lax.broadcasted_iota(jnp.int32, (N, N), 0))
    Wn = graph_W_ref[:, :C]
    Wc = graph_W_ref[:, C:]
    gb = graph_b_ref[...]
    BIG = jnp.float32(2 * N)

    def one_batch(b):
        x = feats_ref[pl.ds(b, 1)].reshape(N, C)
        x_hi = x.astype(BF16).astype(F32)
        x_lo = (x - x_hi).astype(BF16)
        x2b = jnp.concatenate([x_hi.astype(BF16), x_lo], axis=1)
        xxT = _mm_t(x, x)
        sq_col = jnp.sum(x * x, axis=1, keepdims=True)
        sq_row = jnp.sum(jnp.where(eye, sq_col, 0.0), axis=0, keepdims=True)
        score0 = sq_row - 2.0 * xxT
        P_c = _mm_t(x, Wc) + gb
        Wn_b = Wn.astype(BF16)

        def select(score):
            m = jnp.min(score, axis=1, keepdims=True)
            cand = jnp.where(score <= m, jrow_f, BIG)
            jmin = jnp.min(cand, axis=1, keepdims=True)
            onehot = jrow_f == jmin
            return onehot.astype(BF16), jnp.where(onehot, jnp.inf, score)

        def accum4(ohs, M, Sa, Qa):
            oh4 = jnp.concatenate(ohs, axis=0)
            nbr2 = jax.lax.dot_general(oh4, x2b, (((1,), (0,)), ((), ())),
                                       preferred_element_type=F32)
            nbr = nbr2[:, :C] + nbr2[:, C:]
            hs = []
            for i in range(4):
                edge = (nbr[i * N:(i + 1) * N] - x).astype(BF16)
                hs.append(jax.lax.dot_general(
                    edge, Wn_b, (((1,), (1,)), ((), ())),
                    preferred_element_type=F32) + P_c)
            M = jnp.maximum(M, jnp.maximum(jnp.maximum(hs[0], hs[1]),
                                           jnp.maximum(hs[2], hs[3])))
            Sa = Sa + ((hs[0] + hs[1]) + (hs[2] + hs[3]))
            Qa = Qa + ((hs[0] * hs[0] + hs[1] * hs[1])
                       + (hs[2] * hs[2] + hs[3] * hs[3]))
            return M, Sa, Qa

        def select4(score):
            ohs = []
            for _ in range(4):
                oh, score = select(score)
                ohs.append(oh)
            return ohs, score

        M = jnp.full((N, HID), -jnp.inf, F32)
        Sa = jnp.zeros((N, HID), F32)
        Qa = jnp.zeros((N, HID), F32)
        score = score0
        for _ in range(K_NN // 4):
            ohs, score = select4(score)
            M, Sa, Qa = accum4(ohs, M, Sa, Qa)
        scr_ref[pl.ds(b, 1)] = M.reshape(1, N, HID)
        return _colsum(Sa), _colsum(Qa)

    def p1_body(i, carry):
        S1, Q1 = carry
        s0, q0 = one_batch(2 * i)
        s1, q1 = one_batch(2 * i + 1)
        return S1 + (s0 + s1), Q1 + (q0 + q1)

    zero_row = jnp.zeros((1, HID), F32)
    S1, Q1 = jax.lax.fori_loop(0, B // 2, p1_body, (zero_row, zero_row))

    cnt1 = float(B * N * K_NN)
    mean1 = S1 / cnt1
    var1 = Q1 / cnt1 - mean1 * mean1
    sc1 = graph_g_ref[...] / jnp.sqrt(var1 + EPS)
    sh1 = graph_be_ref[...] - mean1 * sc1

    Wf_g = fuse_W_ref[:, :HID]
    Wf_l = fuse_W_ref[:, HID:]

    Mall = scr_ref[...].reshape(BN_ALL, HID)
    t = Mall * sc1 + sh1
    go = jnp.maximum(t, 0.2 * t)
    lf = _mm_t(lang_ref[...], Wf_l)
    lf_exp = jnp.broadcast_to(lf.reshape(B, 1, HID),
                              (B, N, HID)).reshape(BN_ALL, HID)
    ob = obj_ref[...].reshape(BN_ALL, 2)
    maskf = (ob[:, 1:2] > ob[:, 0:1]).astype(F32)
    f = jnp.maximum(_mm_t(go, Wf_g) + lf_exp + fuse_b_ref[...], 0.0) * maskf
    m1 = jnp.maximum(_mm_t(f, w1_ref[...]) + b1_ref[...], 0.0)

    cnt2 = float(BN_ALL)
    mean2 = _colsum(m1) / cnt2
    var2 = _colsum(m1 * m1) / cnt2 - mean2 * mean2
    sc2 = g1_ref[...] / jnp.sqrt(var2 + EPS)
    sh2 = be1_ref[...] - mean2 * sc2

    m2 = jnp.maximum(_mm_t(m1 * sc2 + sh2, w2_ref[...]) + b2_ref[...], 0.0)
    mean3 = _colsum(m2) / cnt2
    var3 = _colsum(m2 * m2) / cnt2 - mean3 * mean3
    sc3 = g2_ref[...] / jnp.sqrt(var3 + EPS)
    sh3 = be2_ref[...] - mean3 * sc3

    m2n = m2 * sc3 + sh3
    conf = _mm_t(w3_ref[...], m2n) + b3_ref[0, 0]
    out_ref[...] = conf.reshape(B, N)


def kernel(aggregated_vote_features, objectness_scores, lang_emb,
           graph_W, graph_b, graph_gamma, graph_beta,
           fuse_W, fuse_b,
           match_W1, match_b1, match_g1, match_be1,
           match_W2, match_b2, match_g2, match_be2,
           match_W3, match_b3):
    row = lambda v: v.reshape(1, -1)
    return pl.pallas_call(
        _fused_body,
        out_shape=jax.ShapeDtypeStruct((B, N), F32),
        scratch_shapes=[pltpu.VMEM((B, N, HID), F32)],
    )(aggregated_vote_features, objectness_scores, lang_emb,
      graph_W, row(graph_b), row(graph_gamma), row(graph_beta),
      fuse_W, row(fuse_b),
      match_W1, row(match_b1), row(match_g1), row(match_be1),
      match_W2, row(match_b2), row(match_g2), row(match_be2),
      match_W3, row(match_b3))
